# baseline (speedup 1.0000x reference)
import functools

import jax
import jax.numpy as jnp
from jax import lax
from jax.experimental import pallas as pl
from jax.experimental.pallas import tpu as pltpu
from jax.experimental.pallas import tpu_sc as plsc

B = 4096
D = 32
GROUP = 128 // D
NROWS = 1000000

_info = plsc.get_sparse_core_info()
_NC, _NS = _info.num_cores, _info.num_subcores
_NW = _NC * _NS
_BPW = B // _NW

_mesh = plsc.VectorSubcoreMesh(core_axis_name="c", subcore_axis_name="s")


@functools.partial(
    pl.kernel,
    mesh=_mesh,
    out_type=[
        jax.ShapeDtypeStruct((B, 128), jnp.float32),
        jax.ShapeDtypeStruct((B, 128), jnp.float32),
    ],
    scratch_types=[
        pltpu.VMEM((_BPW,), jnp.int32),
        pltpu.VMEM((_BPW, 128), jnp.float32),
        pltpu.VMEM((_BPW,), jnp.int32),
        pltpu.VMEM((_BPW, 128), jnp.float32),
        pltpu.SemaphoreType.DMA,
        pltpu.SemaphoreType.DMA,
    ],
)
def _sc_gather(u4_hbm, i4_hbm, ut_hbm, it_hbm, w_hbm, h_hbm,
               uidx_v, urows_v, iidx_v, irows_v, sem_u, sem_i):
    wid = lax.axis_index("s") * _NC + lax.axis_index("c")
    base = wid * _BPW
    pltpu.sync_copy(u4_hbm.at[pl.ds(base, _BPW)], uidx_v)
    pltpu.sync_copy(i4_hbm.at[pl.ds(base, _BPW)], iidx_v)
    cp_u = pltpu.async_copy(ut_hbm.at[uidx_v], urows_v, sem_u)
    cp_i = pltpu.async_copy(it_hbm.at[iidx_v], irows_v, sem_i)
    cp_u.wait()
    cp_i.wait()
    pltpu.sync_copy(urows_v, w_hbm.at[pl.ds(base, _BPW)])
    pltpu.sync_copy(irows_v, h_hbm.at[pl.ds(base, _BPW)])


_BM = 512


def _compact(rows_pad, mod):
    acc = None
    for c in range(GROUP):
        part = rows_pad[:, c * D:(c + 1) * D]
        sel = (mod == c).astype(jnp.float32)
        term = part * sel
        acc = term if acc is None else acc + term
    return acc


def _mm_body(hp_ref, imod_ref, wp_ref, umod_ref, out_ref):
    h = _compact(hp_ref[...], imod_ref[...])
    w = _compact(wp_ref[...], umod_ref[...])
    out_ref[...] = lax.dot_general(
        h, w, (((1,), (1,)), ((), ())),
        preferred_element_type=jnp.float32,
    )


def _tc_matmul(h_pad, i_mod, w_pad, u_mod):
    return pl.pallas_call(
        _mm_body,
        grid=(B // _BM,),
        in_specs=[
            pl.BlockSpec((_BM, 128), lambda m: (m, 0)),
            pl.BlockSpec((_BM, 1), lambda m: (m, 0)),
            pl.BlockSpec((B, 128), lambda m: (0, 0)),
            pl.BlockSpec((B, 1), lambda m: (0, 0)),
        ],
        out_specs=pl.BlockSpec((_BM, B), lambda m: (m, 0)),
        out_shape=jax.ShapeDtypeStruct((B, B), jnp.float32),
    )(h_pad, i_mod, w_pad, u_mod)


@jax.jit
def kernel(u, i, user_table, item_table):
    ut = user_table.reshape(NROWS // GROUP, 128)
    it = item_table.reshape(NROWS // GROUP, 128)
    u4 = u // GROUP
    i4 = i // GROUP
    u_mod = (u % GROUP).astype(jnp.int32).reshape(B, 1)
    i_mod = (i % GROUP).astype(jnp.int32).reshape(B, 1)
    w_pad, h_pad = _sc_gather(u4, i4, ut, it)
    return _tc_matmul(h_pad, i_mod, w_pad, u_mod)


# speedup vs baseline: 8.4797x; 8.4797x over previous
import functools

import jax
import jax.numpy as jnp
from jax import lax
from jax.experimental import pallas as pl
from jax.experimental.pallas import tpu as pltpu
from jax.experimental.pallas import tpu_sc as plsc

B = 4096
D = 32
LANE = 128

_info = plsc.get_sparse_core_info()
_NC, _NS = _info.num_cores, _info.num_subcores
_NW = _NC * _NS
_BPW = B // _NW
_CH = 16

_mesh = plsc.VectorSubcoreMesh(core_axis_name="c", subcore_axis_name="s")


@functools.partial(
    pl.kernel,
    mesh=_mesh,
    out_type=[
        jax.ShapeDtypeStruct((D, B), jnp.float32),
        jax.ShapeDtypeStruct((D, B), jnp.float32),
    ],
    scratch_types=[
        pltpu.VMEM((_BPW,), jnp.int32),
        pltpu.VMEM((_BPW,), jnp.int32),
        pltpu.VMEM((_CH, D, LANE), jnp.float32),
        pltpu.VMEM((D, _BPW), jnp.float32),
        pltpu.VMEM((D, _BPW), jnp.float32),
        pltpu.SemaphoreType.DMA,
    ],
    compiler_params=pltpu.CompilerParams(
        use_tc_tiling_on_sc=True, needs_layout_passes=False
    ),
)
def _sc_gather(u_hbm, i_hbm, utT_hbm, itT_hbm, wT_hbm, hT_hbm,
               u_vm, i_vm, slab_v, w_v, h_v, sem):
    wid = lax.axis_index("s") * _NC + lax.axis_index("c")
    base = wid * _BPW
    pltpu.sync_copy(u_hbm.at[pl.ds(base, _BPW)], u_vm)
    pltpu.sync_copy(i_hbm.at[pl.ds(base, _BPW)], i_vm)

    rows0 = lax.iota(jnp.int32, 16)
    rows1 = rows0 + 16

    def gather_one(table_hbm, idx_vm, dst_v):
        def chunk_body(c, _):
            idx_chunk = idx_vm[pl.ds(c * _CH, _CH)]
            for k in range(_CH):
                idx = idx_chunk[k]
                col0 = pl.multiple_of((idx >> 7) * LANE, LANE)
                pltpu.async_copy(
                    table_hbm.at[:, pl.ds(col0, LANE)],
                    slab_v.at[k],
                    sem,
                )
            for k in range(_CH):
                pltpu.make_async_copy(
                    table_hbm.at[:, pl.ds(0, LANE)],
                    slab_v.at[k],
                    sem,
                ).wait()
            for k in range(_CH):
                idx = idx_chunk[k]
                r = jnp.broadcast_to(idx & (LANE - 1), (16,))
                jcol = jnp.broadcast_to(c * _CH + k, (16,))
                g0 = plsc.load_gather(slab_v.at[k], [rows0, r])
                g1 = plsc.load_gather(slab_v.at[k], [rows1, r])
                plsc.store_scatter(dst_v, [rows0, jcol], g0)
                plsc.store_scatter(dst_v, [rows1, jcol], g1)
            return _

        lax.fori_loop(0, _BPW // _CH, chunk_body, None)

    gather_one(utT_hbm, u_vm, w_v)
    gather_one(itT_hbm, i_vm, h_v)
    pltpu.sync_copy(w_v, wT_hbm.at[:, pl.ds(base, _BPW)])
    pltpu.sync_copy(h_v, hT_hbm.at[:, pl.ds(base, _BPW)])


_BM = 512


def _mm_body(hT_ref, wT_ref, out_ref):
    out_ref[...] = lax.dot_general(
        hT_ref[...], wT_ref[...],
        (((0,), (0,)), ((), ())),
        preferred_element_type=jnp.float32,
    )


def _tc_matmul(hT, wT):
    return pl.pallas_call(
        _mm_body,
        grid=(B // _BM,),
        in_specs=[
            pl.BlockSpec((D, _BM), lambda m: (0, m)),
            pl.BlockSpec((D, B), lambda m: (0, 0)),
        ],
        out_specs=pl.BlockSpec((_BM, B), lambda m: (m, 0)),
        out_shape=jax.ShapeDtypeStruct((B, B), jnp.float32),
    )(hT, wT)


@jax.jit
def kernel(u, i, user_table, item_table):
    utT = user_table.T
    itT = item_table.T
    wT, hT = _sc_gather(u, i, utT, itT)
    return _tc_matmul(hT, wT)


# speedup vs baseline: 8.7671x; 1.0339x over previous
import functools

import jax
import jax.numpy as jnp
from jax import lax
from jax.experimental import pallas as pl
from jax.experimental.pallas import tpu as pltpu
from jax.experimental.pallas import tpu_sc as plsc

B = 4096
D = 32
LANE = 128

_info = plsc.get_sparse_core_info()
_NC, _NS = _info.num_cores, _info.num_subcores
_NW = _NC * _NS
_BPW = B // _NW
_CH = 8

_mesh = plsc.VectorSubcoreMesh(core_axis_name="c", subcore_axis_name="s")


@functools.partial(
    pl.kernel,
    mesh=_mesh,
    out_type=[
        jax.ShapeDtypeStruct((D, B), jnp.float32),
        jax.ShapeDtypeStruct((D, B), jnp.float32),
    ],
    scratch_types=[
        pltpu.VMEM((_BPW,), jnp.int32),
        pltpu.VMEM((_BPW,), jnp.int32),
        pltpu.VMEM((2, _CH, D, LANE), jnp.float32),
        pltpu.VMEM((D, _BPW), jnp.float32),
        pltpu.VMEM((D, _BPW), jnp.float32),
        pltpu.SemaphoreType.DMA,
        pltpu.SemaphoreType.DMA,
    ],
    compiler_params=pltpu.CompilerParams(
        use_tc_tiling_on_sc=True, needs_layout_passes=False
    ),
)
def _sc_gather(u_hbm, i_hbm, utT_hbm, itT_hbm, wT_hbm, hT_hbm,
               u_vm, i_vm, slab_v, w_v, h_v, sem0, sem1):
    wid = lax.axis_index("s") * _NC + lax.axis_index("c")
    base = wid * _BPW
    pltpu.sync_copy(u_hbm.at[pl.ds(base, _BPW)], u_vm)
    pltpu.sync_copy(i_hbm.at[pl.ds(base, _BPW)], i_vm)

    rows0 = lax.iota(jnp.int32, 16)
    rows1 = rows0 + 16

    def gather_one(table_hbm, idx_vm, dst_v):
        nch = _BPW // _CH

        def fire(idx16, off, buf, sem):
            for k in range(_CH):
                idx = idx16[off + k]
                col0 = pl.multiple_of((idx >> 7) * LANE, LANE)
                pltpu.async_copy(
                    table_hbm.at[:, pl.ds(col0, LANE)],
                    slab_v.at[buf, k],
                    sem,
                )

        def drain_extract(idx16, off, c, buf, sem):
            for k in range(_CH):
                pltpu.make_async_copy(
                    table_hbm.at[:, pl.ds(0, LANE)],
                    slab_v.at[buf, k],
                    sem,
                ).wait()
            for k in range(_CH):
                idx = idx16[off + k]
                r = jnp.broadcast_to(idx & (LANE - 1), (16,))
                jcol = jnp.broadcast_to(c * _CH + k, (16,))
                g0 = plsc.load_gather(slab_v.at[buf, k], [rows0, r])
                g1 = plsc.load_gather(slab_v.at[buf, k], [rows1, r])
                plsc.store_scatter(dst_v, [rows0, jcol], g0)
                plsc.store_scatter(dst_v, [rows1, jcol], g1)

        fire(idx_vm[pl.ds(0, 16)], 0, 0, sem0)

        def body(c2, _):
            c = c2 * 2
            idx16 = idx_vm[pl.ds(c2 * 2 * _CH, 2 * _CH)]
            fire(idx16, _CH, 1, sem1)
            drain_extract(idx16, 0, c, 0, sem0)

            @pl.when(c2 < nch // 2 - 1)
            def _fire_next():
                idx16n = idx_vm[pl.ds((c2 + 1) * 2 * _CH, 2 * _CH)]
                fire(idx16n, 0, 0, sem0)

            drain_extract(idx16, _CH, c + 1, 1, sem1)
            return _

        lax.fori_loop(0, nch // 2, body, None)

    gather_one(utT_hbm, u_vm, w_v)
    gather_one(itT_hbm, i_vm, h_v)
    pltpu.sync_copy(w_v, wT_hbm.at[:, pl.ds(base, _BPW)])
    pltpu.sync_copy(h_v, hT_hbm.at[:, pl.ds(base, _BPW)])


_BM = 512


def _mm_body(hT_ref, wT_ref, out_ref):
    out_ref[...] = lax.dot_general(
        hT_ref[...], wT_ref[...],
        (((0,), (0,)), ((), ())),
        preferred_element_type=jnp.float32,
    )


def _tc_matmul(hT, wT):
    return pl.pallas_call(
        _mm_body,
        grid=(B // _BM,),
        in_specs=[
            pl.BlockSpec((D, _BM), lambda m: (0, m)),
            pl.BlockSpec((D, B), lambda m: (0, 0)),
        ],
        out_specs=pl.BlockSpec((_BM, B), lambda m: (m, 0)),
        out_shape=jax.ShapeDtypeStruct((B, B), jnp.float32),
    )(hT, wT)


@jax.jit
def kernel(u, i, user_table, item_table):
    utT = user_table.T
    itT = item_table.T
    wT, hT = _sc_gather(u, i, utT, itT)
    return _tc_matmul(hT, wT)
